# baseline (speedup 1.0000x reference)
import functools

import jax
import jax.numpy as jnp
from jax import lax
from jax.experimental import pallas as pl
from jax.experimental.pallas import tpu as pltpu
from jax.experimental.pallas import tpu_sc as plsc

B = 128
W = 32768
H = 8
K = 64

TILE = 2048
GRID = W // TILE

NC = 2
NS = 16
L = 16
NWORK = NC * NS
ROWS_PER = B // NWORK
NV = W // L


_DN_CONTRACT_MINOR = (((1,), (1,)), ((), ()))


def _mlp_body(x_ref, w1_ref, b1_ref, w2_ref, b2_ref, h2_ref, acc_ref):
    k = pl.program_id(0)

    @pl.when(k == 0)
    def _init():
        acc_ref[...] = jnp.zeros_like(acc_ref)

    acc_ref[...] += lax.dot_general(
        x_ref[...], w1_ref[...], _DN_CONTRACT_MINOR,
        preferred_element_type=jnp.float32)

    @pl.when(k == pl.num_programs(0) - 1)
    def _fin():
        h1 = jnp.maximum(acc_ref[...] + b1_ref[...], 0.0)
        h2 = jnp.maximum(
            lax.dot_general(h1, w2_ref[...], _DN_CONTRACT_MINOR,
                            preferred_element_type=jnp.float32)
            + b2_ref[...], 0.0)
        h2_ref[...] = h2


def _logits_body(h2_ref, w3_ref, b3_ref, z_ref, m_out_ref, m_ref):
    j = pl.program_id(0)
    nj = pl.num_programs(0)
    inf = jnp.float32(jnp.inf)

    z = lax.dot_general(h2_ref[...], w3_ref[...], _DN_CONTRACT_MINOR,
                        preferred_element_type=jnp.float32) + b3_ref[...]
    col = lax.broadcasted_iota(jnp.int32, (B, TILE), 1)
    z = jnp.where((j == 0) & (col == 0), inf, z)
    z = jnp.where((j == nj - 1) & (col == TILE - 1), inf, z)
    z_ref[...] = z

    stepmax = z[:, 0:128]
    for g in range(1, TILE // 128):
        stepmax = jnp.maximum(stepmax, z[:, g * 128:(g + 1) * 128])

    @pl.when(j == 0)
    def _first():
        m_ref[...] = stepmax

    @pl.when(j > 0)
    def _rest():
        m_ref[...] = jnp.maximum(m_ref[...], stepmax)

    @pl.when(j == nj - 1)
    def _fin():
        m_out_ref[...] = m_ref[...]


def _compute_h2(x, w1, b1r, w2, b2r):
    return pl.pallas_call(
        _mlp_body,
        grid=(GRID,),
        in_specs=[
            pl.BlockSpec((B, TILE), lambda k: (0, k)),
            pl.BlockSpec((H, TILE), lambda k: (0, k)),
            pl.BlockSpec((1, H), lambda k: (0, 0)),
            pl.BlockSpec((H, H), lambda k: (0, 0)),
            pl.BlockSpec((1, H), lambda k: (0, 0)),
        ],
        out_specs=pl.BlockSpec((B, H), lambda k: (0, 0)),
        out_shape=jax.ShapeDtypeStruct((B, H), jnp.float32),
        scratch_shapes=[pltpu.VMEM((B, H), jnp.float32)],
    )(x, w1, b1r, w2, b2r)


def _compute_logits(h2, w3, b3r):
    return pl.pallas_call(
        _logits_body,
        grid=(GRID,),
        in_specs=[
            pl.BlockSpec((B, H), lambda j: (0, 0)),
            pl.BlockSpec((TILE, H), lambda j: (j, 0)),
            pl.BlockSpec((1, TILE), lambda j: (0, j)),
        ],
        out_specs=[
            pl.BlockSpec((B, TILE), lambda j: (0, j)),
            pl.BlockSpec((B, 128), lambda j: (0, 0)),
        ],
        out_shape=[
            jax.ShapeDtypeStruct((B, W), jnp.float32),
            jax.ShapeDtypeStruct((B, 128), jnp.float32),
        ],
        scratch_shapes=[pltpu.VMEM((B, 128), jnp.float32)],
    )(h2, w3, b3r)


_MASK31 = 0x7FFFFFFF
_INT_MIN = -2147483648


def _to_keys(ref, base, nv):
    mask31 = jnp.int32(_MASK31)

    def key_body(i, carry):
        v = ref[pl.ds(base + i * L, L)]
        bits = plsc.bitcast(v, jnp.int32)
        kk = bits ^ (lax.shift_right_arithmetic(bits, 31) & mask31)
        ref[pl.ds(base + i * L, L)] = plsc.bitcast(kk, jnp.float32)
        return carry

    lax.fori_loop(0, nv, key_body, jnp.int32(0))


def _bsearch_kth(ref, base, nv, k_target):
    ktv = jnp.full((L,), k_target, jnp.int32)
    mask31 = jnp.int32(_MASK31)

    def count_ge(candb):
        def cbody(i, acc):
            kv = plsc.bitcast(ref[pl.ds(base + i * L, L)], jnp.int32)
            return acc + plsc.all_reduce_population_count(kv >= candb)
        return lax.fori_loop(0, nv, cbody, jnp.zeros((L,), jnp.int32))

    pos = count_ge(jnp.zeros((L,), jnp.int32)) >= ktv
    bse = jnp.where(pos, jnp.zeros((L,), jnp.int32),
                    jnp.full((L,), _INT_MIN, jnp.int32))

    def bit_body(i, bse):
        bitv = lax.shift_left(jnp.full((L,), 1, jnp.int32),
                              jnp.full((L,), 30, jnp.int32) - i)
        cand = bse | bitv
        return jnp.where(count_ge(cand) >= ktv, cand, bse)

    bse = lax.fori_loop(0, 31, bit_body, bse)
    bits = jnp.where(bse < 0, bse ^ mask31, bse)
    return plsc.bitcast(bits, jnp.float32)


def _sc_body(z_hbm, m_hbm, thr_hbm, row0_v, row1_v, cand_v, ids_v, m_v,
             stage_v, sem0, sem1):
    wid = lax.axis_index("s") * NC + lax.axis_index("c")
    bufs = (row0_v, row1_v)
    sems = (sem0, sem1)
    r0 = wid * ROWS_PER
    descs = [pltpu.async_copy(z_hbm.at[r0], row0_v, sem0), None]
    for k in range(ROWS_PER):
        pltpu.sync_copy(m_hbm.at[r0 + k], m_v.at[pl.ds(k * 128, 128)])
    _to_keys(m_v, 0, ROWS_PER * 128 // L)

    for k in range(ROWS_PER):
        r = r0 + k
        buf = bufs[k % 2]
        t0b = _bsearch_kth(m_v, k * 128, 128 // L, K)
        descs[k % 2].wait()
        if k + 1 < ROWS_PER:
            descs[(k + 1) % 2] = pltpu.async_copy(
                z_hbm.at[r + 1], bufs[(k + 1) % 2], sems[(k + 1) % 2])
        lane = lax.iota(jnp.int32, L)

        def flag_body(i, nf_vec):
            cvec = jnp.zeros((L,), jnp.int32)
            for u in range(L):
                v = buf[pl.ds((i * L + u) * L, L)]
                p = plsc.all_reduce_population_count(v >= t0b)
                cvec = jnp.where(lane == u, p, cvec)
            mk = cvec > 0
            cs = plsc.cumsum(mk.astype(jnp.int32))
            idx = jnp.where(mk, nf_vec + cs - 1, 0)
            plsc.store_scatter(ids_v, [idx], i * L + lane, mask=mk)
            return nf_vec + plsc.all_reduce_population_count(mk)

        nf_vec = lax.fori_loop(0, NV // L, flag_body,
                               jnp.zeros((L,), jnp.int32))
        nflag = jnp.max(nf_vec)

        def scan_body(j, ptr_vec):
            vid = ids_v[pl.ds(j, L)][0]
            v = buf[pl.ds(vid * L, L)]
            mk = v >= t0b
            cs = plsc.cumsum(mk.astype(jnp.int32))
            idx = jnp.where(mk, ptr_vec + cs - 1, 0)
            plsc.store_scatter(cand_v, [idx], v, mask=mk)
            return ptr_vec + plsc.all_reduce_population_count(mk)

        ptr_vec = lax.fori_loop(0, nflag, scan_body,
                                jnp.zeros((L,), jnp.int32))
        c = jnp.max(ptr_vec)
        pad_idx = c + lax.iota(jnp.int32, L)
        plsc.store_scatter(cand_v, [pad_idx],
                           jnp.full((L,), -jnp.inf, jnp.float32))
        nv = (c + (L - 1)) // L

        _to_keys(cand_v, 0, nv)
        stage_v[...] = _bsearch_kth(cand_v, 0, nv, K)
        pltpu.sync_copy(stage_v, thr_hbm.at[r])


_sc_select = functools.partial(
    pl.kernel,
    out_type=jax.ShapeDtypeStruct((B, L), jnp.float32),
    mesh=plsc.VectorSubcoreMesh(core_axis_name="c", subcore_axis_name="s"),
    compiler_params=pltpu.CompilerParams(needs_layout_passes=False,
                                         use_tc_tiling_on_sc=True),
    scratch_types=[
        pltpu.VMEM((W,), jnp.float32),
        pltpu.VMEM((W,), jnp.float32),
        pltpu.VMEM((W + L,), jnp.float32),
        pltpu.VMEM((NV + L,), jnp.int32),
        pltpu.VMEM((ROWS_PER * 128 + L,), jnp.float32),
        pltpu.VMEM((L,), jnp.float32),
        pltpu.SemaphoreType.DMA,
        pltpu.SemaphoreType.DMA,
    ],
)(_sc_body)


def _mask_body(h2_ref, w3_ref, b3_ref, thr_ref, y_ref):
    j = pl.program_id(0)
    nj = pl.num_programs(0)
    inf = jnp.float32(jnp.inf)
    z = lax.dot_general(h2_ref[...], w3_ref[...], _DN_CONTRACT_MINOR,
                        preferred_element_type=jnp.float32) + b3_ref[...]
    col = lax.broadcasted_iota(jnp.int32, (B, TILE), 1)
    z = jnp.where((j == 0) & (col == 0), inf, z)
    z = jnp.where((j == nj - 1) & (col == TILE - 1), inf, z)
    th = thr_ref[:, 0:1]
    y_ref[...] = jnp.where(z >= th, 1.0, 0.0)


def _apply_mask(h2, w3, b3r, thr):
    return pl.pallas_call(
        _mask_body,
        grid=(GRID,),
        in_specs=[
            pl.BlockSpec((B, H), lambda j: (0, 0)),
            pl.BlockSpec((TILE, H), lambda j: (j, 0)),
            pl.BlockSpec((1, TILE), lambda j: (0, j)),
            pl.BlockSpec((B, L), lambda j: (0, 0)),
        ],
        out_specs=pl.BlockSpec((B, TILE), lambda j: (0, j)),
        out_shape=jax.ShapeDtypeStruct((B, W), jnp.float32),
    )(h2, w3, b3r, thr)


@jax.jit
def kernel(x, W1, b1, W2, b2, W3, b3):
    b1r = b1.reshape(1, H)
    b2r = b2.reshape(1, H)
    b3r = b3.reshape(1, W)

    h2 = _compute_h2(x, W1, b1r, W2, b2r)
    z, m = _compute_logits(h2, W3, b3r)
    thr = _sc_select(z, m)
    return _apply_mask(h2, W3, b3r, thr)


# speedup vs baseline: 1.0060x; 1.0060x over previous
import functools

import jax
import jax.numpy as jnp
from jax import lax
from jax.experimental import pallas as pl
from jax.experimental.pallas import tpu as pltpu
from jax.experimental.pallas import tpu_sc as plsc

B = 128
W = 32768
H = 8
K = 64

TILE = 2048
GRID = W // TILE

NC = 2
NS = 16
L = 16
NWORK = NC * NS
ROWS_PER = B // NWORK
NV = W // L


_DN_CONTRACT_MINOR = (((1,), (1,)), ((), ()))


def _mlp_body(x_ref, w1_ref, b1_ref, w2_ref, b2_ref, h2_ref, acc_ref):
    k = pl.program_id(0)

    @pl.when(k == 0)
    def _init():
        acc_ref[...] = jnp.zeros_like(acc_ref)

    acc_ref[...] += lax.dot_general(
        x_ref[...], w1_ref[...], _DN_CONTRACT_MINOR,
        preferred_element_type=jnp.float32)

    @pl.when(k == pl.num_programs(0) - 1)
    def _fin():
        h1 = jnp.maximum(acc_ref[...] + b1_ref[...], 0.0)
        h2 = jnp.maximum(
            lax.dot_general(h1, w2_ref[...], _DN_CONTRACT_MINOR,
                            preferred_element_type=jnp.float32)
            + b2_ref[...], 0.0)
        h2_ref[...] = h2


def _logits_body(h2_ref, w3_ref, b3_ref, z_ref, m128_ref):
    j = pl.program_id(0)
    nj = pl.num_programs(0)
    inf = jnp.float32(jnp.inf)

    z = lax.dot_general(h2_ref[...], w3_ref[...], _DN_CONTRACT_MINOR,
                        preferred_element_type=jnp.float32) + b3_ref[...]
    col = lax.broadcasted_iota(jnp.int32, (B, TILE), 1)
    z = jnp.where((j == 0) & (col == 0), inf, z)
    z = jnp.where((j == nj - 1) & (col == TILE - 1), inf, z)
    z_ref[...] = z

    cm = jnp.max(z.reshape(B, TILE // 128, 128), axis=2)
    m128_ref[...] = cm.reshape(1, B, TILE // 128)


def _compute_h2(x, w1, b1r, w2, b2r):
    return pl.pallas_call(
        _mlp_body,
        grid=(GRID,),
        in_specs=[
            pl.BlockSpec((B, TILE), lambda k: (0, k)),
            pl.BlockSpec((H, TILE), lambda k: (0, k)),
            pl.BlockSpec((1, H), lambda k: (0, 0)),
            pl.BlockSpec((H, H), lambda k: (0, 0)),
            pl.BlockSpec((1, H), lambda k: (0, 0)),
        ],
        out_specs=pl.BlockSpec((B, H), lambda k: (0, 0)),
        out_shape=jax.ShapeDtypeStruct((B, H), jnp.float32),
        scratch_shapes=[pltpu.VMEM((B, H), jnp.float32)],
    )(x, w1, b1r, w2, b2r)


def _compute_logits(h2, w3, b3r):
    return pl.pallas_call(
        _logits_body,
        grid=(GRID,),
        in_specs=[
            pl.BlockSpec((B, H), lambda j: (0, 0)),
            pl.BlockSpec((TILE, H), lambda j: (j, 0)),
            pl.BlockSpec((1, TILE), lambda j: (0, j)),
        ],
        out_specs=[
            pl.BlockSpec((B, TILE), lambda j: (0, j)),
            pl.BlockSpec((1, B, TILE // 128), lambda j: (j, 0, 0)),
        ],
        out_shape=[
            jax.ShapeDtypeStruct((B, W), jnp.float32),
            jax.ShapeDtypeStruct((GRID, B, TILE // 128), jnp.float32),
        ],
    )(h2, w3, b3r)


_MASK31 = 0x7FFFFFFF
_INT_MIN = -2147483648


def _to_keys(ref, base, nv):
    mask31 = jnp.int32(_MASK31)

    def key_body(i, carry):
        v = ref[pl.ds(base + i * L, L)]
        bits = plsc.bitcast(v, jnp.int32)
        kk = bits ^ (lax.shift_right_arithmetic(bits, 31) & mask31)
        ref[pl.ds(base + i * L, L)] = plsc.bitcast(kk, jnp.float32)
        return carry

    lax.fori_loop(0, nv, key_body, jnp.int32(0))


def _bsearch_kth_regs(kvs, k_target, npasses):
    ktv = jnp.full((L,), k_target, jnp.int32)

    def count_ge(candb):
        acc = jnp.zeros((L,), jnp.int32)
        for kv in kvs:
            acc = acc + plsc.all_reduce_population_count(kv >= candb)
        return acc

    pos = count_ge(jnp.zeros((L,), jnp.int32)) >= ktv
    bse = jnp.where(pos, jnp.zeros((L,), jnp.int32),
                    jnp.full((L,), _INT_MIN, jnp.int32))

    def bit_body(i, bse):
        bitv = lax.shift_left(jnp.full((L,), 1, jnp.int32),
                              jnp.full((L,), 30, jnp.int32) - i)
        cand = bse | bitv
        return jnp.where(count_ge(cand) >= ktv, cand, bse)

    return lax.fori_loop(0, npasses, bit_body, bse)


def _bsearch_kth(ref, base, nv, k_target):
    ktv = jnp.full((L,), k_target, jnp.int32)
    mask31 = jnp.int32(_MASK31)

    def count_ge(candb):
        def cbody(i, acc):
            kv = plsc.bitcast(ref[pl.ds(base + i * L, L)], jnp.int32)
            return acc + plsc.all_reduce_population_count(kv >= candb)
        return lax.fori_loop(0, nv, cbody, jnp.zeros((L,), jnp.int32))

    pos = count_ge(jnp.zeros((L,), jnp.int32)) >= ktv
    bse = jnp.where(pos, jnp.zeros((L,), jnp.int32),
                    jnp.full((L,), _INT_MIN, jnp.int32))

    def bit_body(i, bse):
        bitv = lax.shift_left(jnp.full((L,), 1, jnp.int32),
                              jnp.full((L,), 30, jnp.int32) - i)
        cand = bse | bitv
        return jnp.where(count_ge(cand) >= ktv, cand, bse)

    bse = lax.fori_loop(0, 31, bit_body, bse)
    bits = jnp.where(bse < 0, bse ^ mask31, bse)
    return plsc.bitcast(bits, jnp.float32)


_CPR = TILE // 128


def _sc_body(z_hbm, m128_hbm, thr_hbm, row0_v, row1_v, cand_v, ids_v, cids_v,
             m_v, stage_v, sem0, sem1, semm):
    wid = lax.axis_index("s") * NC + lax.axis_index("c")
    bufs = (row0_v, row1_v)
    sems = (sem0, sem1)
    r0 = wid * ROWS_PER
    mask31 = jnp.int32(_MASK31)
    descs = [pltpu.async_copy(z_hbm.at[r0], row0_v, sem0), None]
    mdescs = []
    for k in range(ROWS_PER):
        for j in range(GRID):
            mdescs.append(pltpu.async_copy(
                m128_hbm.at[j, pl.ds((r0 + k) * _CPR, _CPR)],
                m_v.at[pl.ds((k * GRID + j) * L, L)], semm))

    for k in range(ROWS_PER):
        r = r0 + k
        buf = bufs[k % 2]
        lane = lax.iota(jnp.int32, L)
        for j in range(GRID):
            mdescs[k * GRID + j].wait()
        mvregs = [m_v[pl.ds((k * GRID + g) * L, L)] for g in range(GRID)]
        m8 = [jnp.maximum(mvregs[g], mvregs[g + 8]) for g in range(8)]
        m8k = []
        for x in m8:
            bits = plsc.bitcast(x, jnp.int32)
            m8k.append(bits ^ (lax.shift_right_arithmetic(bits, 31) & mask31))
        t0key = _bsearch_kth_regs(m8k, K, 17)
        t0bits = jnp.where(t0key < 0, t0key ^ mask31, t0key)
        t0b = plsc.bitcast(t0bits, jnp.float32)

        nc_vec = jnp.zeros((L,), jnp.int32)
        for i in range(GRID):
            mk = mvregs[i] >= t0b
            cs = plsc.cumsum(mk.astype(jnp.int32))
            idx = jnp.where(mk, nc_vec + cs - 1, 0)
            plsc.store_scatter(cids_v, [idx], i * L + lane, mask=mk)
            nc_vec = nc_vec + plsc.all_reduce_population_count(mk)
        nchunk = jnp.max(nc_vec)

        descs[k % 2].wait()
        if k + 1 < ROWS_PER:
            descs[(k + 1) % 2] = pltpu.async_copy(
                z_hbm.at[r + 1], bufs[(k + 1) % 2], sems[(k + 1) % 2])

        def cflag_body(j, nf_vec):
            cid = cids_v[pl.ds(j, L)][0]
            cvec = jnp.zeros((L,), jnp.int32)
            for u in range(8):
                v = buf[pl.ds(cid * 128 + u * L, L)]
                p = plsc.all_reduce_population_count(v >= t0b)
                cvec = jnp.where(lane == u, p, cvec)
            mk = cvec > 0
            cs = plsc.cumsum(mk.astype(jnp.int32))
            idx = jnp.where(mk, nf_vec + cs - 1, 0)
            plsc.store_scatter(ids_v, [idx], cid * 8 + lane, mask=mk)
            return nf_vec + plsc.all_reduce_population_count(mk)

        nf_vec = lax.fori_loop(0, nchunk, cflag_body,
                               jnp.zeros((L,), jnp.int32))
        nflag = jnp.max(nf_vec)

        def scan_body(j, ptr_vec):
            vid = ids_v[pl.ds(j, L)][0]
            v = buf[pl.ds(vid * L, L)]
            mk = v >= t0b
            cs = plsc.cumsum(mk.astype(jnp.int32))
            idx = jnp.where(mk, ptr_vec + cs - 1, 0)
            plsc.store_scatter(cand_v, [idx], v, mask=mk)
            return ptr_vec + plsc.all_reduce_population_count(mk)

        ptr_vec = lax.fori_loop(0, nflag, scan_body,
                                jnp.zeros((L,), jnp.int32))
        c = jnp.max(ptr_vec)
        pad_idx = c + lax.iota(jnp.int32, L)
        plsc.store_scatter(cand_v, [pad_idx],
                           jnp.full((L,), -jnp.inf, jnp.float32))
        nv = (c + (L - 1)) // L

        _to_keys(cand_v, 0, nv)
        stage_v[...] = _bsearch_kth(cand_v, 0, nv, K)
        pltpu.sync_copy(stage_v, thr_hbm.at[r])


_sc_select = functools.partial(
    pl.kernel,
    out_type=jax.ShapeDtypeStruct((B, L), jnp.float32),
    mesh=plsc.VectorSubcoreMesh(core_axis_name="c", subcore_axis_name="s"),
    compiler_params=pltpu.CompilerParams(needs_layout_passes=False),
    scratch_types=[
        pltpu.VMEM((W,), jnp.float32),
        pltpu.VMEM((W,), jnp.float32),
        pltpu.VMEM((W + L,), jnp.float32),
        pltpu.VMEM((NV + L,), jnp.int32),
        pltpu.VMEM((GRID * _CPR + L,), jnp.int32),
        pltpu.VMEM((ROWS_PER * GRID * L,), jnp.float32),
        pltpu.VMEM((L,), jnp.float32),
        pltpu.SemaphoreType.DMA,
        pltpu.SemaphoreType.DMA,
        pltpu.SemaphoreType.DMA,
    ],
)(_sc_body)


def _mask_body(h2_ref, w3_ref, b3_ref, thr_ref, y_ref):
    j = pl.program_id(0)
    nj = pl.num_programs(0)
    inf = jnp.float32(jnp.inf)
    z = lax.dot_general(h2_ref[...], w3_ref[...], _DN_CONTRACT_MINOR,
                        preferred_element_type=jnp.float32) + b3_ref[...]
    col = lax.broadcasted_iota(jnp.int32, (B, TILE), 1)
    z = jnp.where((j == 0) & (col == 0), inf, z)
    z = jnp.where((j == nj - 1) & (col == TILE - 1), inf, z)
    th = thr_ref[:, 0:1]
    y_ref[...] = jnp.where(z >= th, 1.0, 0.0)


def _apply_mask(h2, w3, b3r, thr):
    return pl.pallas_call(
        _mask_body,
        grid=(GRID,),
        in_specs=[
            pl.BlockSpec((B, H), lambda j: (0, 0)),
            pl.BlockSpec((TILE, H), lambda j: (j, 0)),
            pl.BlockSpec((1, TILE), lambda j: (0, j)),
            pl.BlockSpec((B, L), lambda j: (0, 0)),
        ],
        out_specs=pl.BlockSpec((B, TILE), lambda j: (0, j)),
        out_shape=jax.ShapeDtypeStruct((B, W), jnp.float32),
    )(h2, w3, b3r, thr)


@jax.jit
def kernel(x, W1, b1, W2, b2, W3, b3):
    b1r = b1.reshape(1, H)
    b2r = b2.reshape(1, H)
    b3r = b3.reshape(1, W)

    h2 = _compute_h2(x, W1, b1r, W2, b2r)
    z, m128 = _compute_logits(h2, W3, b3r)
    thr = _sc_select(z, m128.reshape(GRID, B * _CPR))
    return _apply_mask(h2, W3, b3r, thr)


# speedup vs baseline: 1.2306x; 1.2233x over previous
import functools

import jax
import jax.numpy as jnp
from jax import lax
from jax.experimental import pallas as pl
from jax.experimental.pallas import tpu as pltpu
from jax.experimental.pallas import tpu_sc as plsc

B = 128
W = 32768
H = 8
K = 64

TILE = 2048
GRID = W // TILE

NC = 2
NS = 16
L = 16
NWORK = NC * NS
ROWS_PER = B // NWORK
NV = W // L


_DN_CONTRACT_MINOR = (((1,), (1,)), ((), ()))


def _mlp_body(x_ref, w1_ref, b1_ref, w2_ref, b2_ref, h2_ref, acc_ref):
    k = pl.program_id(0)

    @pl.when(k == 0)
    def _init():
        acc_ref[...] = jnp.zeros_like(acc_ref)

    acc_ref[...] += lax.dot_general(
        x_ref[...], w1_ref[...], _DN_CONTRACT_MINOR,
        preferred_element_type=jnp.float32)

    @pl.when(k == pl.num_programs(0) - 1)
    def _fin():
        h1 = jnp.maximum(acc_ref[...] + b1_ref[...], 0.0)
        h2 = jnp.maximum(
            lax.dot_general(h1, w2_ref[...], _DN_CONTRACT_MINOR,
                            preferred_element_type=jnp.float32)
            + b2_ref[...], 0.0)
        h2_ref[...] = h2


def _z_tile(h2_ref, w3t_ref, b3_ref, j, nj):
    inf = jnp.float32(jnp.inf)
    z = jnp.dot(h2_ref[...], w3t_ref[...],
                preferred_element_type=jnp.float32) + b3_ref[...]
    col = lax.broadcasted_iota(jnp.int32, (B, TILE), 1)
    z = jnp.where((j == 0) & (col == 0), inf, z)
    z = jnp.where((j == nj - 1) & (col == TILE - 1), inf, z)
    return z


def _logits_body(h2_ref, w3t_ref, b3_ref, z_ref, m128_ref):
    j = pl.program_id(0)
    nj = pl.num_programs(0)
    z = _z_tile(h2_ref, w3t_ref, b3_ref, j, nj)
    z_ref[...] = z

    cm = jnp.max(z.reshape(B, TILE // 128, 128), axis=2)
    m128_ref[...] = cm.reshape(1, B, TILE // 128)


def _compute_h2(x, w1, b1r, w2, b2r):
    return pl.pallas_call(
        _mlp_body,
        grid=(GRID,),
        in_specs=[
            pl.BlockSpec((B, TILE), lambda k: (0, k)),
            pl.BlockSpec((H, TILE), lambda k: (0, k)),
            pl.BlockSpec((1, H), lambda k: (0, 0)),
            pl.BlockSpec((H, H), lambda k: (0, 0)),
            pl.BlockSpec((1, H), lambda k: (0, 0)),
        ],
        out_specs=pl.BlockSpec((B, H), lambda k: (0, 0)),
        out_shape=jax.ShapeDtypeStruct((B, H), jnp.float32),
        scratch_shapes=[pltpu.VMEM((B, H), jnp.float32)],
    )(x, w1, b1r, w2, b2r)


def _compute_logits(h2, w3t, b3r):
    return pl.pallas_call(
        _logits_body,
        grid=(GRID,),
        in_specs=[
            pl.BlockSpec((B, H), lambda j: (0, 0)),
            pl.BlockSpec((H, TILE), lambda j: (0, j)),
            pl.BlockSpec((1, TILE), lambda j: (0, j)),
        ],
        out_specs=[
            pl.BlockSpec((B, TILE), lambda j: (0, j)),
            pl.BlockSpec((1, B, TILE // 128), lambda j: (j, 0, 0)),
        ],
        out_shape=[
            jax.ShapeDtypeStruct((B, W), jnp.float32),
            jax.ShapeDtypeStruct((GRID, B, TILE // 128), jnp.float32),
        ],
    )(h2, w3t, b3r)


_MASK31 = 0x7FFFFFFF
_INT_MIN = -2147483648


def _to_keys(ref, base, nv):
    mask31 = jnp.int32(_MASK31)

    def key_body(i, carry):
        v = ref[pl.ds(base + i * L, L)]
        bits = plsc.bitcast(v, jnp.int32)
        kk = bits ^ (lax.shift_right_arithmetic(bits, 31) & mask31)
        ref[pl.ds(base + i * L, L)] = plsc.bitcast(kk, jnp.float32)
        return carry

    lax.fori_loop(0, nv, key_body, jnp.int32(0))


def _bsearch_kth_regs(kvs, k_target, npasses):
    ktv = jnp.full((L,), k_target, jnp.int32)

    def count_ge(candb):
        acc = jnp.zeros((L,), jnp.int32)
        for kv in kvs:
            acc = acc + plsc.all_reduce_population_count(kv >= candb)
        return acc

    pos = count_ge(jnp.zeros((L,), jnp.int32)) >= ktv
    bse = jnp.where(pos, jnp.zeros((L,), jnp.int32),
                    jnp.full((L,), _INT_MIN, jnp.int32))

    def bit_body(i, bse):
        bitv = lax.shift_left(jnp.full((L,), 1, jnp.int32),
                              jnp.full((L,), 30, jnp.int32) - i)
        cand = bse | bitv
        return jnp.where(count_ge(cand) >= ktv, cand, bse)

    return lax.fori_loop(0, npasses, bit_body, bse)


def _bsearch_kth(ref, base, nv, k_target):
    ktv = jnp.full((L,), k_target, jnp.int32)
    mask31 = jnp.int32(_MASK31)

    def count_ge(candb):
        def cbody(i, acc):
            kv = plsc.bitcast(ref[pl.ds(base + i * L, L)], jnp.int32)
            return acc + plsc.all_reduce_population_count(kv >= candb)
        return lax.fori_loop(0, nv, cbody, jnp.zeros((L,), jnp.int32))

    pos = count_ge(jnp.zeros((L,), jnp.int32)) >= ktv
    bse = jnp.where(pos, jnp.zeros((L,), jnp.int32),
                    jnp.full((L,), _INT_MIN, jnp.int32))

    def bit_body(i, bse):
        bitv = lax.shift_left(jnp.full((L,), 1, jnp.int32),
                              jnp.full((L,), 30, jnp.int32) - i)
        cand = bse | bitv
        return jnp.where(count_ge(cand) >= ktv, cand, bse)

    bse = lax.fori_loop(0, 31, bit_body, bse)
    bits = jnp.where(bse < 0, bse ^ mask31, bse)
    return plsc.bitcast(bits, jnp.float32)


_CPR = TILE // 128


def _sc_body(z_hbm, m128_hbm, thr_hbm, row0_v, row1_v, cand_v, ids_v, cids_v,
             m_v, stage_v, sem0, sem1, semm):
    wid = lax.axis_index("s") * NC + lax.axis_index("c")
    bufs = (row0_v, row1_v)
    sems = (sem0, sem1)
    r0 = wid * ROWS_PER
    mask31 = jnp.int32(_MASK31)
    descs = [pltpu.async_copy(z_hbm.at[r0], row0_v, sem0), None]
    mdescs = []
    for k in range(ROWS_PER):
        for j in range(GRID):
            mdescs.append(pltpu.async_copy(
                m128_hbm.at[j, pl.ds((r0 + k) * _CPR, _CPR)],
                m_v.at[pl.ds((k * GRID + j) * L, L)], semm))

    for k in range(ROWS_PER):
        r = r0 + k
        buf = bufs[k % 2]
        lane = lax.iota(jnp.int32, L)
        for j in range(GRID):
            mdescs[k * GRID + j].wait()
        mvregs = [m_v[pl.ds((k * GRID + g) * L, L)] for g in range(GRID)]
        m8 = [jnp.maximum(mvregs[g], mvregs[g + 8]) for g in range(8)]
        m8k = []
        for x in m8:
            bits = plsc.bitcast(x, jnp.int32)
            m8k.append(bits ^ (lax.shift_right_arithmetic(bits, 31) & mask31))
        t0key = _bsearch_kth_regs(m8k, K, 17)
        t0bits = jnp.where(t0key < 0, t0key ^ mask31, t0key)
        t0b = plsc.bitcast(t0bits, jnp.float32)

        nc_vec = jnp.zeros((L,), jnp.int32)
        for i in range(GRID):
            mk = mvregs[i] >= t0b
            cs = plsc.cumsum(mk.astype(jnp.int32))
            idx = jnp.where(mk, nc_vec + cs - 1, 0)
            plsc.store_scatter(cids_v, [idx], i * L + lane, mask=mk)
            nc_vec = nc_vec + plsc.all_reduce_population_count(mk)
        nchunk = jnp.max(nc_vec)

        descs[k % 2].wait()
        if k + 1 < ROWS_PER:
            descs[(k + 1) % 2] = pltpu.async_copy(
                z_hbm.at[r + 1], bufs[(k + 1) % 2], sems[(k + 1) % 2])

        def cflag_body(j, nf_vec):
            cid = cids_v[pl.ds(j, L)][0]
            cvec = jnp.zeros((L,), jnp.int32)
            for u in range(8):
                v = buf[pl.ds(cid * 128 + u * L, L)]
                p = plsc.all_reduce_population_count(v >= t0b)
                cvec = jnp.where(lane == u, p, cvec)
            mk = cvec > 0
            cs = plsc.cumsum(mk.astype(jnp.int32))
            idx = jnp.where(mk, nf_vec + cs - 1, 0)
            plsc.store_scatter(ids_v, [idx], cid * 8 + lane, mask=mk)
            return nf_vec + plsc.all_reduce_population_count(mk)

        nf_vec = lax.fori_loop(0, nchunk, cflag_body,
                               jnp.zeros((L,), jnp.int32))
        nflag = jnp.max(nf_vec)

        def scan_body(j, ptr_vec):
            vid = ids_v[pl.ds(j, L)][0]
            v = buf[pl.ds(vid * L, L)]
            mk = v >= t0b
            cs = plsc.cumsum(mk.astype(jnp.int32))
            idx = jnp.where(mk, ptr_vec + cs - 1, 0)
            plsc.store_scatter(cand_v, [idx], v, mask=mk)
            return ptr_vec + plsc.all_reduce_population_count(mk)

        ptr_vec = lax.fori_loop(0, nflag, scan_body,
                                jnp.zeros((L,), jnp.int32))
        c = jnp.max(ptr_vec)
        pad_idx = c + lax.iota(jnp.int32, L)
        plsc.store_scatter(cand_v, [pad_idx],
                           jnp.full((L,), -jnp.inf, jnp.float32))
        nv = (c + (L - 1)) // L

        _to_keys(cand_v, 0, nv)
        stage_v[...] = _bsearch_kth(cand_v, 0, nv, K)
        pltpu.sync_copy(stage_v, thr_hbm.at[r])


_sc_select = functools.partial(
    pl.kernel,
    out_type=jax.ShapeDtypeStruct((B, L), jnp.float32),
    mesh=plsc.VectorSubcoreMesh(core_axis_name="c", subcore_axis_name="s"),
    compiler_params=pltpu.CompilerParams(needs_layout_passes=False),
    scratch_types=[
        pltpu.VMEM((W,), jnp.float32),
        pltpu.VMEM((W,), jnp.float32),
        pltpu.VMEM((W + L,), jnp.float32),
        pltpu.VMEM((NV + L,), jnp.int32),
        pltpu.VMEM((GRID * _CPR + L,), jnp.int32),
        pltpu.VMEM((ROWS_PER * GRID * L,), jnp.float32),
        pltpu.VMEM((L,), jnp.float32),
        pltpu.SemaphoreType.DMA,
        pltpu.SemaphoreType.DMA,
        pltpu.SemaphoreType.DMA,
    ],
)(_sc_body)


def _mask_body(h2_ref, w3t_ref, b3_ref, thr_ref, y_ref):
    j = pl.program_id(0)
    nj = pl.num_programs(0)
    z = _z_tile(h2_ref, w3t_ref, b3_ref, j, nj)
    th = thr_ref[:, 0:1]
    y_ref[...] = jnp.where(z >= th, 1.0, 0.0)


def _apply_mask(h2, w3t, b3r, thr):
    return pl.pallas_call(
        _mask_body,
        grid=(GRID,),
        in_specs=[
            pl.BlockSpec((B, H), lambda j: (0, 0)),
            pl.BlockSpec((H, TILE), lambda j: (0, j)),
            pl.BlockSpec((1, TILE), lambda j: (0, j)),
            pl.BlockSpec((B, L), lambda j: (0, 0)),
        ],
        out_specs=pl.BlockSpec((B, TILE), lambda j: (0, j)),
        out_shape=jax.ShapeDtypeStruct((B, W), jnp.float32),
    )(h2, w3t, b3r, thr)


@jax.jit
def kernel(x, W1, b1, W2, b2, W3, b3):
    b1r = b1.reshape(1, H)
    b2r = b2.reshape(1, H)
    b3r = b3.reshape(1, W)
    w3t = W3.T

    h2 = _compute_h2(x, W1, b1r, W2, b2r)
    z, m128 = _compute_logits(h2, w3t, b3r)
    thr = _sc_select(z, m128.reshape(GRID, B * _CPR))
    return _apply_mask(h2, w3t, b3r, thr)


# speedup vs baseline: 1.2928x; 1.0506x over previous
import functools

import jax
import jax.numpy as jnp
from jax import lax
from jax.experimental import pallas as pl
from jax.experimental.pallas import tpu as pltpu
from jax.experimental.pallas import tpu_sc as plsc

B = 128
W = 32768
H = 8
K = 64

TILE = 2048
GRID = W // TILE

NC = 2
NS = 16
L = 16
NWORK = NC * NS
ROWS_PER = B // NWORK
NV = W // L


_DN_CONTRACT_MINOR = (((1,), (1,)), ((), ()))


def _mlp_body(x_ref, w1_ref, b1_ref, w2_ref, b2_ref, h2_ref, acc_ref):
    k = pl.program_id(0)

    @pl.when(k == 0)
    def _init():
        acc_ref[...] = jnp.zeros_like(acc_ref)

    acc_ref[...] += lax.dot_general(
        x_ref[...], w1_ref[...], _DN_CONTRACT_MINOR,
        preferred_element_type=jnp.float32)

    @pl.when(k == pl.num_programs(0) - 1)
    def _fin():
        h1 = jnp.maximum(acc_ref[...] + b1_ref[...], 0.0)
        h2 = jnp.maximum(
            lax.dot_general(h1, w2_ref[...], _DN_CONTRACT_MINOR,
                            preferred_element_type=jnp.float32)
            + b2_ref[...], 0.0)
        h2_ref[...] = h2


def _z_tile(h2_ref, w3t_ref, b3_ref, j, nj):
    inf = jnp.float32(jnp.inf)
    z = jnp.dot(h2_ref[...], w3t_ref[...],
                preferred_element_type=jnp.float32) + b3_ref[...]
    col = lax.broadcasted_iota(jnp.int32, (B, TILE), 1)
    z = jnp.where((j == 0) & (col == 0), inf, z)
    z = jnp.where((j == nj - 1) & (col == TILE - 1), inf, z)
    return z


def _logits_body(h2_ref, w3t_ref, b3_ref, z_ref, m128_ref):
    j = pl.program_id(0)
    nj = pl.num_programs(0)
    z = _z_tile(h2_ref, w3t_ref, b3_ref, j, nj)
    z_ref[...] = z

    cm = jnp.max(z.reshape(B, TILE // 128, 128), axis=2)
    m128_ref[...] = cm.reshape(1, B, TILE // 128)


_TILE_A1 = 4096


def _compute_h2(x, w1, b1r, w2, b2r):
    return pl.pallas_call(
        _mlp_body,
        grid=(W // _TILE_A1,),
        in_specs=[
            pl.BlockSpec((B, _TILE_A1), lambda k: (0, k)),
            pl.BlockSpec((H, _TILE_A1), lambda k: (0, k)),
            pl.BlockSpec((1, H), lambda k: (0, 0)),
            pl.BlockSpec((H, H), lambda k: (0, 0)),
            pl.BlockSpec((1, H), lambda k: (0, 0)),
        ],
        out_specs=pl.BlockSpec((B, H), lambda k: (0, 0)),
        out_shape=jax.ShapeDtypeStruct((B, H), jnp.float32),
        scratch_shapes=[pltpu.VMEM((B, H), jnp.float32)],
    )(x, w1, b1r, w2, b2r)


def _compute_logits(h2, w3t, b3r):
    return pl.pallas_call(
        _logits_body,
        grid=(GRID,),
        in_specs=[
            pl.BlockSpec((B, H), lambda j: (0, 0)),
            pl.BlockSpec((H, TILE), lambda j: (0, j)),
            pl.BlockSpec((1, TILE), lambda j: (0, j)),
        ],
        out_specs=[
            pl.BlockSpec((B, TILE), lambda j: (0, j)),
            pl.BlockSpec((1, B, TILE // 128), lambda j: (j, 0, 0)),
        ],
        out_shape=[
            jax.ShapeDtypeStruct((B, W), jnp.float32),
            jax.ShapeDtypeStruct((GRID, B, TILE // 128), jnp.float32),
        ],
    )(h2, w3t, b3r)


_MASK31 = 0x7FFFFFFF
_INT_MIN = -2147483648


def _to_keys(ref, base, nv):
    mask31 = jnp.int32(_MASK31)

    def key_body(i, carry):
        v = ref[pl.ds(base + i * L, L)]
        bits = plsc.bitcast(v, jnp.int32)
        kk = bits ^ (lax.shift_right_arithmetic(bits, 31) & mask31)
        ref[pl.ds(base + i * L, L)] = plsc.bitcast(kk, jnp.float32)
        return carry

    lax.fori_loop(0, nv, key_body, jnp.int32(0))


def _bsearch_kth_regs(kvs, k_target, npasses):
    ktv = jnp.full((L,), k_target, jnp.int32)

    def count_ge(candb):
        acc = jnp.zeros((L,), jnp.int32)
        for kv in kvs:
            acc = acc + plsc.all_reduce_population_count(kv >= candb)
        return acc

    pos = count_ge(jnp.zeros((L,), jnp.int32)) >= ktv
    bse = jnp.where(pos, jnp.zeros((L,), jnp.int32),
                    jnp.full((L,), _INT_MIN, jnp.int32))

    def bit_body(i, bse):
        bitv = lax.shift_left(jnp.full((L,), 1, jnp.int32),
                              jnp.full((L,), 30, jnp.int32) - i)
        cand = bse | bitv
        return jnp.where(count_ge(cand) >= ktv, cand, bse)

    return lax.fori_loop(0, npasses, bit_body, bse)


def _bsearch_kth(ref, base, nv, k_target):
    ktv = jnp.full((L,), k_target, jnp.int32)
    mask31 = jnp.int32(_MASK31)

    def count_ge(candb):
        def cbody(i, acc):
            kv = plsc.bitcast(ref[pl.ds(base + i * L, L)], jnp.int32)
            return acc + plsc.all_reduce_population_count(kv >= candb)
        return lax.fori_loop(0, nv, cbody, jnp.zeros((L,), jnp.int32))

    pos = count_ge(jnp.zeros((L,), jnp.int32)) >= ktv
    bse = jnp.where(pos, jnp.zeros((L,), jnp.int32),
                    jnp.full((L,), _INT_MIN, jnp.int32))

    def bit_body(i, bse):
        bitv = lax.shift_left(jnp.full((L,), 1, jnp.int32),
                              jnp.full((L,), 30, jnp.int32) - i)
        cand = bse | bitv
        return jnp.where(count_ge(cand) >= ktv, cand, bse)

    bse = lax.fori_loop(0, 31, bit_body, bse)
    bits = jnp.where(bse < 0, bse ^ mask31, bse)
    return plsc.bitcast(bits, jnp.float32)


_CPR = TILE // 128


def _sc_body(z_hbm, m128_hbm, thr_hbm, row0_v, row1_v, cand_v, ids_v, cids_v,
             m_v, stage_v, sem0, sem1, semm):
    wid = lax.axis_index("s") * NC + lax.axis_index("c")
    bufs = (row0_v, row1_v)
    sems = (sem0, sem1)
    r0 = wid * ROWS_PER
    mask31 = jnp.int32(_MASK31)
    descs = [pltpu.async_copy(z_hbm.at[r0], row0_v, sem0), None]
    mdescs = []
    for k in range(ROWS_PER):
        for j in range(GRID):
            mdescs.append(pltpu.async_copy(
                m128_hbm.at[j, pl.ds((r0 + k) * _CPR, _CPR)],
                m_v.at[pl.ds((k * GRID + j) * L, L)], semm))

    for k in range(ROWS_PER):
        r = r0 + k
        buf = bufs[k % 2]
        lane = lax.iota(jnp.int32, L)
        for j in range(GRID):
            mdescs[k * GRID + j].wait()
        mvregs = [m_v[pl.ds((k * GRID + g) * L, L)] for g in range(GRID)]
        m8 = [jnp.maximum(mvregs[g], mvregs[g + 8]) for g in range(8)]
        m8k = []
        for x in m8:
            bits = plsc.bitcast(x, jnp.int32)
            m8k.append(bits ^ (lax.shift_right_arithmetic(bits, 31) & mask31))
        t0key = _bsearch_kth_regs(m8k, K, 17)
        t0bits = jnp.where(t0key < 0, t0key ^ mask31, t0key)
        t0b = plsc.bitcast(t0bits, jnp.float32)

        nc_vec = jnp.zeros((L,), jnp.int32)
        for i in range(GRID):
            mk = mvregs[i] >= t0b
            cs = plsc.cumsum(mk.astype(jnp.int32))
            idx = jnp.where(mk, nc_vec + cs - 1, 0)
            plsc.store_scatter(cids_v, [idx], i * L + lane, mask=mk)
            nc_vec = nc_vec + plsc.all_reduce_population_count(mk)
        nchunk = jnp.max(nc_vec)

        descs[k % 2].wait()
        if k + 1 < ROWS_PER:
            descs[(k + 1) % 2] = pltpu.async_copy(
                z_hbm.at[r + 1], bufs[(k + 1) % 2], sems[(k + 1) % 2])

        def cflag_body(j, nf_vec):
            cid = cids_v[pl.ds(j, L)][0]
            cvec = jnp.zeros((L,), jnp.int32)
            for u in range(8):
                v = buf[pl.ds(cid * 128 + u * L, L)]
                p = plsc.all_reduce_population_count(v >= t0b)
                cvec = jnp.where(lane == u, p, cvec)
            mk = cvec > 0
            cs = plsc.cumsum(mk.astype(jnp.int32))
            idx = jnp.where(mk, nf_vec + cs - 1, 0)
            plsc.store_scatter(ids_v, [idx], cid * 8 + lane, mask=mk)
            return nf_vec + plsc.all_reduce_population_count(mk)

        nf_vec = lax.fori_loop(0, nchunk, cflag_body,
                               jnp.zeros((L,), jnp.int32))
        nflag = jnp.max(nf_vec)

        def scan_body(j, ptr_vec):
            vid = ids_v[pl.ds(j, L)][0]
            v = buf[pl.ds(vid * L, L)]
            mk = v >= t0b
            cs = plsc.cumsum(mk.astype(jnp.int32))
            idx = jnp.where(mk, ptr_vec + cs - 1, 0)
            plsc.store_scatter(cand_v, [idx], v, mask=mk)
            return ptr_vec + plsc.all_reduce_population_count(mk)

        ptr_vec = lax.fori_loop(0, nflag, scan_body,
                                jnp.zeros((L,), jnp.int32))
        c = jnp.max(ptr_vec)
        pad_idx = c + lax.iota(jnp.int32, L)
        plsc.store_scatter(cand_v, [pad_idx],
                           jnp.full((L,), -jnp.inf, jnp.float32))
        nv = (c + (L - 1)) // L

        _to_keys(cand_v, 0, nv)
        stage_v[...] = _bsearch_kth(cand_v, 0, nv, K)
        pltpu.sync_copy(stage_v, thr_hbm.at[r])


_sc_select = functools.partial(
    pl.kernel,
    out_type=jax.ShapeDtypeStruct((B, L), jnp.float32),
    mesh=plsc.VectorSubcoreMesh(core_axis_name="c", subcore_axis_name="s"),
    compiler_params=pltpu.CompilerParams(needs_layout_passes=False),
    scratch_types=[
        pltpu.VMEM((W,), jnp.float32),
        pltpu.VMEM((W,), jnp.float32),
        pltpu.VMEM((W + L,), jnp.float32),
        pltpu.VMEM((NV + L,), jnp.int32),
        pltpu.VMEM((GRID * _CPR + L,), jnp.int32),
        pltpu.VMEM((ROWS_PER * GRID * L,), jnp.float32),
        pltpu.VMEM((L,), jnp.float32),
        pltpu.SemaphoreType.DMA,
        pltpu.SemaphoreType.DMA,
        pltpu.SemaphoreType.DMA,
    ],
)(_sc_body)


def _mask_body(h2_ref, w3t_ref, b3_ref, thr_ref, y_ref):
    j = pl.program_id(0)
    nj = pl.num_programs(0)
    z = _z_tile(h2_ref, w3t_ref, b3_ref, j, nj)
    th = thr_ref[:, 0:1]
    y_ref[...] = jnp.where(z >= th, 1.0, 0.0)


def _apply_mask(h2, w3t, b3r, thr):
    return pl.pallas_call(
        _mask_body,
        grid=(GRID,),
        in_specs=[
            pl.BlockSpec((B, H), lambda j: (0, 0)),
            pl.BlockSpec((H, TILE), lambda j: (0, j)),
            pl.BlockSpec((1, TILE), lambda j: (0, j)),
            pl.BlockSpec((B, L), lambda j: (0, 0)),
        ],
        out_specs=pl.BlockSpec((B, TILE), lambda j: (0, j)),
        out_shape=jax.ShapeDtypeStruct((B, W), jnp.float32),
    )(h2, w3t, b3r, thr)


@jax.jit
def kernel(x, W1, b1, W2, b2, W3, b3):
    b1r = b1.reshape(1, H)
    b2r = b2.reshape(1, H)
    b3r = b3.reshape(1, W)
    w3t = W3.T

    h2 = _compute_h2(x, W1, b1r, W2, b2r)
    z, m128 = _compute_logits(h2, w3t, b3r)
    thr = _sc_select(z, m128.reshape(GRID, B * _CPR))
    return _apply_mask(h2, w3t, b3r, thr)


# speedup vs baseline: 1.4460x; 1.1185x over previous
import functools

import jax
import jax.numpy as jnp
from jax import lax
from jax.experimental import pallas as pl
from jax.experimental.pallas import tpu as pltpu
from jax.experimental.pallas import tpu_sc as plsc

B = 128
W = 32768
H = 8
K = 64

TILE = 4096
GRID = W // TILE

NC = 2
NS = 16
L = 16
NWORK = NC * NS
ROWS_PER = B // NWORK
NV = W // L


_DN_CONTRACT_MINOR = (((1,), (1,)), ((), ()))


def _mlp_body(x_ref, w1_ref, b1_ref, w2_ref, b2_ref, h2_ref, acc_ref):
    k = pl.program_id(0)

    @pl.when(k == 0)
    def _init():
        acc_ref[...] = jnp.zeros_like(acc_ref)

    acc_ref[...] += lax.dot_general(
        x_ref[...], w1_ref[...], _DN_CONTRACT_MINOR,
        preferred_element_type=jnp.float32)

    @pl.when(k == pl.num_programs(0) - 1)
    def _fin():
        h1 = jnp.maximum(acc_ref[...] + b1_ref[...], 0.0)
        h2 = jnp.maximum(
            lax.dot_general(h1, w2_ref[...], _DN_CONTRACT_MINOR,
                            preferred_element_type=jnp.float32)
            + b2_ref[...], 0.0)
        h2_ref[...] = h2


def _z_tile(h2_ref, w3t_ref, b3_ref, j, nj):
    inf = jnp.float32(jnp.inf)
    z = jnp.dot(h2_ref[...], w3t_ref[...],
                preferred_element_type=jnp.float32) + b3_ref[...]
    col = lax.broadcasted_iota(jnp.int32, (B, TILE), 1)
    z = jnp.where((j == 0) & (col == 0), inf, z)
    z = jnp.where((j == nj - 1) & (col == TILE - 1), inf, z)
    return z


def _logits_body(h2_ref, w3t_ref, b3_ref, z_ref, m128_ref):
    j = pl.program_id(0)
    nj = pl.num_programs(0)
    z = _z_tile(h2_ref, w3t_ref, b3_ref, j, nj)
    z_ref[...] = z

    cm = jnp.max(z.reshape(B, TILE // 128, 128), axis=2)
    m128_ref[...] = cm.reshape(1, B, TILE // 128)


_TILE_A1 = 4096


def _compute_h2(x, w1, b1r, w2, b2r):
    return pl.pallas_call(
        _mlp_body,
        grid=(W // _TILE_A1,),
        in_specs=[
            pl.BlockSpec((B, _TILE_A1), lambda k: (0, k)),
            pl.BlockSpec((H, _TILE_A1), lambda k: (0, k)),
            pl.BlockSpec((1, H), lambda k: (0, 0)),
            pl.BlockSpec((H, H), lambda k: (0, 0)),
            pl.BlockSpec((1, H), lambda k: (0, 0)),
        ],
        out_specs=pl.BlockSpec((B, H), lambda k: (0, 0)),
        out_shape=jax.ShapeDtypeStruct((B, H), jnp.float32),
        scratch_shapes=[pltpu.VMEM((B, H), jnp.float32)],
    )(x, w1, b1r, w2, b2r)


def _compute_logits(h2, w3t, b3r):
    return pl.pallas_call(
        _logits_body,
        grid=(GRID,),
        in_specs=[
            pl.BlockSpec((B, H), lambda j: (0, 0)),
            pl.BlockSpec((H, TILE), lambda j: (0, j)),
            pl.BlockSpec((1, TILE), lambda j: (0, j)),
        ],
        out_specs=[
            pl.BlockSpec((B, TILE), lambda j: (0, j)),
            pl.BlockSpec((1, B, TILE // 128), lambda j: (j, 0, 0)),
        ],
        out_shape=[
            jax.ShapeDtypeStruct((B, W), jnp.float32),
            jax.ShapeDtypeStruct((GRID, B, TILE // 128), jnp.float32),
        ],
    )(h2, w3t, b3r)


_MASK31 = 0x7FFFFFFF
_INT_MIN = -2147483648


def _to_keys(ref, base, nv):
    mask31 = jnp.int32(_MASK31)

    def key_body(i, carry):
        v = ref[pl.ds(base + i * L, L)]
        bits = plsc.bitcast(v, jnp.int32)
        kk = bits ^ (lax.shift_right_arithmetic(bits, 31) & mask31)
        ref[pl.ds(base + i * L, L)] = plsc.bitcast(kk, jnp.float32)
        return carry

    lax.fori_loop(0, nv, key_body, jnp.int32(0))


def _bsearch_kth_regs(kvs, k_target, npasses):
    ktv = jnp.full((L,), k_target, jnp.int32)

    def count_ge(candb):
        acc = jnp.zeros((L,), jnp.int32)
        for kv in kvs:
            acc = acc + plsc.all_reduce_population_count(kv >= candb)
        return acc

    pos = count_ge(jnp.zeros((L,), jnp.int32)) >= ktv
    bse = jnp.where(pos, jnp.zeros((L,), jnp.int32),
                    jnp.full((L,), _INT_MIN, jnp.int32))

    def bit_body(i, bse):
        bitv = lax.shift_left(jnp.full((L,), 1, jnp.int32),
                              jnp.full((L,), 30, jnp.int32) - i)
        cand = bse | bitv
        return jnp.where(count_ge(cand) >= ktv, cand, bse)

    return lax.fori_loop(0, npasses, bit_body, bse)


def _bsearch_kth(ref, base, nv, k_target):
    ktv = jnp.full((L,), k_target, jnp.int32)
    mask31 = jnp.int32(_MASK31)

    def count_ge(candb):
        def cbody(i, acc):
            kv = plsc.bitcast(ref[pl.ds(base + i * L, L)], jnp.int32)
            return acc + plsc.all_reduce_population_count(kv >= candb)
        return lax.fori_loop(0, nv, cbody, jnp.zeros((L,), jnp.int32))

    pos = count_ge(jnp.zeros((L,), jnp.int32)) >= ktv
    bse = jnp.where(pos, jnp.zeros((L,), jnp.int32),
                    jnp.full((L,), _INT_MIN, jnp.int32))

    def bit_body(i, bse):
        bitv = lax.shift_left(jnp.full((L,), 1, jnp.int32),
                              jnp.full((L,), 30, jnp.int32) - i)
        cand = bse | bitv
        return jnp.where(count_ge(cand) >= ktv, cand, bse)

    bse = lax.fori_loop(0, 31, bit_body, bse)
    bits = jnp.where(bse < 0, bse ^ mask31, bse)
    return plsc.bitcast(bits, jnp.float32)


_CPR = TILE // 128


def _sc_body(z_hbm, m128_hbm, thr_hbm, row0_v, row1_v, cand_v, ids_v, cids_v,
             m_v, stage_v, sem0, sem1, semm):
    wid = lax.axis_index("s") * NC + lax.axis_index("c")
    bufs = (row0_v, row1_v)
    sems = (sem0, sem1)
    r0 = wid * ROWS_PER
    mask31 = jnp.int32(_MASK31)
    descs = [pltpu.async_copy(z_hbm.at[r0], row0_v, sem0), None]
    mdescs = []
    for k in range(ROWS_PER):
        for j in range(GRID):
            mdescs.append(pltpu.async_copy(
                m128_hbm.at[j, pl.ds((r0 + k) * _CPR, _CPR)],
                m_v.at[pl.ds((k * GRID + j) * _CPR, _CPR)], semm))

    for k in range(ROWS_PER):
        r = r0 + k
        buf = bufs[k % 2]
        lane = lax.iota(jnp.int32, L)
        for j in range(GRID):
            mdescs[k * GRID + j].wait()
        mvregs = [m_v[pl.ds(k * (GRID * _CPR) + g * L, L)]
                  for g in range(GRID * _CPR // L)]
        m8 = [jnp.maximum(mvregs[g], mvregs[g + 8]) for g in range(8)]
        m8k = []
        for x in m8:
            bits = plsc.bitcast(x, jnp.int32)
            m8k.append(bits ^ (lax.shift_right_arithmetic(bits, 31) & mask31))
        t0key = _bsearch_kth_regs(m8k, K, 17)
        t0bits = jnp.where(t0key < 0, t0key ^ mask31, t0key)
        t0b = plsc.bitcast(t0bits, jnp.float32)

        nc_vec = jnp.zeros((L,), jnp.int32)
        for i in range(GRID * _CPR // L):
            mk = mvregs[i] >= t0b
            cs = plsc.cumsum(mk.astype(jnp.int32))
            idx = jnp.where(mk, nc_vec + cs - 1, 0)
            plsc.store_scatter(cids_v, [idx], i * L + lane, mask=mk)
            nc_vec = nc_vec + plsc.all_reduce_population_count(mk)
        nchunk = jnp.max(nc_vec)

        descs[k % 2].wait()
        if k + 1 < ROWS_PER:
            descs[(k + 1) % 2] = pltpu.async_copy(
                z_hbm.at[r + 1], bufs[(k + 1) % 2], sems[(k + 1) % 2])

        def cflag_body(j, nf_vec):
            cid = cids_v[pl.ds(j, L)][0]
            cvec = jnp.zeros((L,), jnp.int32)
            for u in range(8):
                v = buf[pl.ds(cid * 128 + u * L, L)]
                p = plsc.all_reduce_population_count(v >= t0b)
                cvec = jnp.where(lane == u, p, cvec)
            mk = cvec > 0
            cs = plsc.cumsum(mk.astype(jnp.int32))
            idx = jnp.where(mk, nf_vec + cs - 1, 0)
            plsc.store_scatter(ids_v, [idx], cid * 8 + lane, mask=mk)
            return nf_vec + plsc.all_reduce_population_count(mk)

        nf_vec = lax.fori_loop(0, nchunk, cflag_body,
                               jnp.zeros((L,), jnp.int32))
        nflag = jnp.max(nf_vec)

        def scan_body(j, ptr_vec):
            vid = ids_v[pl.ds(j, L)][0]
            v = buf[pl.ds(vid * L, L)]
            mk = v >= t0b
            cs = plsc.cumsum(mk.astype(jnp.int32))
            idx = jnp.where(mk, ptr_vec + cs - 1, 0)
            plsc.store_scatter(cand_v, [idx], v, mask=mk)
            return ptr_vec + plsc.all_reduce_population_count(mk)

        ptr_vec = lax.fori_loop(0, nflag, scan_body,
                                jnp.zeros((L,), jnp.int32))
        c = jnp.max(ptr_vec)
        pad_idx = c + lax.iota(jnp.int32, L)
        plsc.store_scatter(cand_v, [pad_idx],
                           jnp.full((L,), -jnp.inf, jnp.float32))
        nv = (c + (L - 1)) // L

        _to_keys(cand_v, 0, nv)
        stage_v[...] = _bsearch_kth(cand_v, 0, nv, K)
        pltpu.sync_copy(stage_v, thr_hbm.at[r])


_sc_select = functools.partial(
    pl.kernel,
    out_type=jax.ShapeDtypeStruct((B, L), jnp.float32),
    mesh=plsc.VectorSubcoreMesh(core_axis_name="c", subcore_axis_name="s"),
    compiler_params=pltpu.CompilerParams(needs_layout_passes=False),
    scratch_types=[
        pltpu.VMEM((W,), jnp.float32),
        pltpu.VMEM((W,), jnp.float32),
        pltpu.VMEM((W + L,), jnp.float32),
        pltpu.VMEM((NV + L,), jnp.int32),
        pltpu.VMEM((GRID * _CPR + L,), jnp.int32),
        pltpu.VMEM((ROWS_PER * GRID * _CPR,), jnp.float32),
        pltpu.VMEM((L,), jnp.float32),
        pltpu.SemaphoreType.DMA,
        pltpu.SemaphoreType.DMA,
        pltpu.SemaphoreType.DMA,
    ],
)(_sc_body)


def _mask_body(h2_ref, w3t_ref, b3_ref, thr_ref, y_ref):
    j = pl.program_id(0)
    nj = pl.num_programs(0)
    z = _z_tile(h2_ref, w3t_ref, b3_ref, j, nj)
    th = thr_ref[:, 0:1]
    y_ref[...] = jnp.where(z >= th, 1.0, 0.0)


def _apply_mask(h2, w3t, b3r, thr):
    return pl.pallas_call(
        _mask_body,
        grid=(GRID,),
        in_specs=[
            pl.BlockSpec((B, H), lambda j: (0, 0)),
            pl.BlockSpec((H, TILE), lambda j: (0, j)),
            pl.BlockSpec((1, TILE), lambda j: (0, j)),
            pl.BlockSpec((B, L), lambda j: (0, 0)),
        ],
        out_specs=pl.BlockSpec((B, TILE), lambda j: (0, j)),
        out_shape=jax.ShapeDtypeStruct((B, W), jnp.float32),
    )(h2, w3t, b3r, thr)


@jax.jit
def kernel(x, W1, b1, W2, b2, W3, b3):
    b1r = b1.reshape(1, H)
    b2r = b2.reshape(1, H)
    b3r = b3.reshape(1, W)
    w3t = W3.T

    h2 = _compute_h2(x, W1, b1r, W2, b2r)
    z, m128 = _compute_logits(h2, w3t, b3r)
    thr = _sc_select(z, m128.reshape(GRID, B * _CPR))
    return _apply_mask(h2, w3t, b3r, thr)


# speedup vs baseline: 1.5428x; 1.0669x over previous
import functools

import jax
import jax.numpy as jnp
from jax import lax
from jax.experimental import pallas as pl
from jax.experimental.pallas import tpu as pltpu
from jax.experimental.pallas import tpu_sc as plsc

B = 128
W = 32768
H = 8
K = 64

TILE = 8192
GRID = W // TILE

NC = 2
NS = 16
L = 16
NWORK = NC * NS
ROWS_PER = B // NWORK
NV = W // L


_DN_CONTRACT_MINOR = (((1,), (1,)), ((), ()))


def _mlp_body(x_ref, w1_ref, b1_ref, w2_ref, b2_ref, h2_ref, acc_ref):
    k = pl.program_id(0)

    @pl.when(k == 0)
    def _init():
        acc_ref[...] = jnp.zeros_like(acc_ref)

    acc_ref[...] += lax.dot_general(
        x_ref[...], w1_ref[...], _DN_CONTRACT_MINOR,
        preferred_element_type=jnp.float32)

    @pl.when(k == pl.num_programs(0) - 1)
    def _fin():
        h1 = jnp.maximum(acc_ref[...] + b1_ref[...], 0.0)
        h2 = jnp.maximum(
            lax.dot_general(h1, w2_ref[...], _DN_CONTRACT_MINOR,
                            preferred_element_type=jnp.float32)
            + b2_ref[...], 0.0)
        h2_ref[...] = h2


def _z_tile(h2_ref, w3t_ref, b3_ref, j, nj):
    inf = jnp.float32(jnp.inf)
    z = jnp.dot(h2_ref[...], w3t_ref[...],
                preferred_element_type=jnp.float32) + b3_ref[...]
    col = lax.broadcasted_iota(jnp.int32, (B, TILE), 1)
    z = jnp.where((j == 0) & (col == 0), inf, z)
    z = jnp.where((j == nj - 1) & (col == TILE - 1), inf, z)
    return z


def _logits_body(h2_ref, w3t_ref, b3_ref, z_ref, m128_ref):
    j = pl.program_id(0)
    nj = pl.num_programs(0)
    z = _z_tile(h2_ref, w3t_ref, b3_ref, j, nj)
    z_ref[...] = z

    cm = jnp.max(z.reshape(B, TILE // 128, 128), axis=2)
    m128_ref[...] = cm.reshape(1, B, TILE // 128)


_TILE_A1 = 8192


def _compute_h2(x, w1, b1r, w2, b2r):
    return pl.pallas_call(
        _mlp_body,
        grid=(W // _TILE_A1,),
        in_specs=[
            pl.BlockSpec((B, _TILE_A1), lambda k: (0, k)),
            pl.BlockSpec((H, _TILE_A1), lambda k: (0, k)),
            pl.BlockSpec((1, H), lambda k: (0, 0)),
            pl.BlockSpec((H, H), lambda k: (0, 0)),
            pl.BlockSpec((1, H), lambda k: (0, 0)),
        ],
        out_specs=pl.BlockSpec((B, H), lambda k: (0, 0)),
        out_shape=jax.ShapeDtypeStruct((B, H), jnp.float32),
        scratch_shapes=[pltpu.VMEM((B, H), jnp.float32)],
    )(x, w1, b1r, w2, b2r)


def _compute_logits(h2, w3t, b3r):
    return pl.pallas_call(
        _logits_body,
        grid=(GRID,),
        in_specs=[
            pl.BlockSpec((B, H), lambda j: (0, 0)),
            pl.BlockSpec((H, TILE), lambda j: (0, j)),
            pl.BlockSpec((1, TILE), lambda j: (0, j)),
        ],
        out_specs=[
            pl.BlockSpec((B, TILE), lambda j: (0, j)),
            pl.BlockSpec((1, B, TILE // 128), lambda j: (j, 0, 0)),
        ],
        out_shape=[
            jax.ShapeDtypeStruct((B, W), jnp.float32),
            jax.ShapeDtypeStruct((GRID, B, TILE // 128), jnp.float32),
        ],
    )(h2, w3t, b3r)


_MASK31 = 0x7FFFFFFF
_INT_MIN = -2147483648


def _to_keys(ref, base, nv):
    mask31 = jnp.int32(_MASK31)

    def key_body(i, carry):
        v = ref[pl.ds(base + i * L, L)]
        bits = plsc.bitcast(v, jnp.int32)
        kk = bits ^ (lax.shift_right_arithmetic(bits, 31) & mask31)
        ref[pl.ds(base + i * L, L)] = plsc.bitcast(kk, jnp.float32)
        return carry

    lax.fori_loop(0, nv, key_body, jnp.int32(0))


def _bsearch_kth_regs(kvs, k_target, npasses):
    ktv = jnp.full((L,), k_target, jnp.int32)

    def count_ge(candb):
        acc = jnp.zeros((L,), jnp.int32)
        for kv in kvs:
            acc = acc + plsc.all_reduce_population_count(kv >= candb)
        return acc

    pos = count_ge(jnp.zeros((L,), jnp.int32)) >= ktv
    bse = jnp.where(pos, jnp.zeros((L,), jnp.int32),
                    jnp.full((L,), _INT_MIN, jnp.int32))

    def bit_body(i, bse):
        bitv = lax.shift_left(jnp.full((L,), 1, jnp.int32),
                              jnp.full((L,), 30, jnp.int32) - i)
        cand = bse | bitv
        return jnp.where(count_ge(cand) >= ktv, cand, bse)

    return lax.fori_loop(0, npasses, bit_body, bse)


def _bsearch_kth(ref, base, nv, k_target):
    ktv = jnp.full((L,), k_target, jnp.int32)
    mask31 = jnp.int32(_MASK31)

    def count_ge(candb):
        def cbody(i, acc):
            kv = plsc.bitcast(ref[pl.ds(base + i * L, L)], jnp.int32)
            return acc + plsc.all_reduce_population_count(kv >= candb)
        return lax.fori_loop(0, nv, cbody, jnp.zeros((L,), jnp.int32))

    pos = count_ge(jnp.zeros((L,), jnp.int32)) >= ktv
    bse = jnp.where(pos, jnp.zeros((L,), jnp.int32),
                    jnp.full((L,), _INT_MIN, jnp.int32))

    def bit_body(i, bse):
        bitv = lax.shift_left(jnp.full((L,), 1, jnp.int32),
                              jnp.full((L,), 30, jnp.int32) - i)
        cand = bse | bitv
        return jnp.where(count_ge(cand) >= ktv, cand, bse)

    bse = lax.fori_loop(0, 31, bit_body, bse)
    bits = jnp.where(bse < 0, bse ^ mask31, bse)
    return plsc.bitcast(bits, jnp.float32)


_CPR = TILE // 128


def _sc_body(z_hbm, m128_hbm, thr_hbm, row0_v, row1_v, cand_v, ids_v, cids_v,
             m_v, stage_v, sem0, sem1, semm):
    wid = lax.axis_index("s") * NC + lax.axis_index("c")
    bufs = (row0_v, row1_v)
    sems = (sem0, sem1)
    r0 = wid * ROWS_PER
    mask31 = jnp.int32(_MASK31)
    descs = [pltpu.async_copy(z_hbm.at[r0], row0_v, sem0), None]
    mdescs = []
    for k in range(ROWS_PER):
        for j in range(GRID):
            mdescs.append(pltpu.async_copy(
                m128_hbm.at[j, pl.ds((r0 + k) * _CPR, _CPR)],
                m_v.at[pl.ds((k * GRID + j) * _CPR, _CPR)], semm))

    for k in range(ROWS_PER):
        r = r0 + k
        buf = bufs[k % 2]
        lane = lax.iota(jnp.int32, L)
        for j in range(GRID):
            mdescs[k * GRID + j].wait()
        mvregs = [m_v[pl.ds(k * (GRID * _CPR) + g * L, L)]
                  for g in range(GRID * _CPR // L)]
        m8 = [jnp.maximum(mvregs[g], mvregs[g + 8]) for g in range(8)]
        m8k = []
        for x in m8:
            bits = plsc.bitcast(x, jnp.int32)
            m8k.append(bits ^ (lax.shift_right_arithmetic(bits, 31) & mask31))
        t0key = _bsearch_kth_regs(m8k, K, 17)
        t0bits = jnp.where(t0key < 0, t0key ^ mask31, t0key)
        t0b = plsc.bitcast(t0bits, jnp.float32)

        nc_vec = jnp.zeros((L,), jnp.int32)
        for i in range(GRID * _CPR // L):
            mk = mvregs[i] >= t0b
            cs = plsc.cumsum(mk.astype(jnp.int32))
            idx = jnp.where(mk, nc_vec + cs - 1, 0)
            plsc.store_scatter(cids_v, [idx], i * L + lane, mask=mk)
            nc_vec = nc_vec + plsc.all_reduce_population_count(mk)
        nchunk = jnp.max(nc_vec)

        descs[k % 2].wait()
        if k + 1 < ROWS_PER:
            descs[(k + 1) % 2] = pltpu.async_copy(
                z_hbm.at[r + 1], bufs[(k + 1) % 2], sems[(k + 1) % 2])

        def cflag_body(j, nf_vec):
            cid = cids_v[pl.ds(j, L)][0]
            cvec = jnp.zeros((L,), jnp.int32)
            for u in range(8):
                v = buf[pl.ds(cid * 128 + u * L, L)]
                p = plsc.all_reduce_population_count(v >= t0b)
                cvec = jnp.where(lane == u, p, cvec)
            mk = cvec > 0
            cs = plsc.cumsum(mk.astype(jnp.int32))
            idx = jnp.where(mk, nf_vec + cs - 1, 0)
            plsc.store_scatter(ids_v, [idx], cid * 8 + lane, mask=mk)
            return nf_vec + plsc.all_reduce_population_count(mk)

        nf_vec = lax.fori_loop(0, nchunk, cflag_body,
                               jnp.zeros((L,), jnp.int32))
        nflag = jnp.max(nf_vec)

        def scan_body(j, ptr_vec):
            vid = ids_v[pl.ds(j, L)][0]
            v = buf[pl.ds(vid * L, L)]
            mk = v >= t0b
            cs = plsc.cumsum(mk.astype(jnp.int32))
            idx = jnp.where(mk, ptr_vec + cs - 1, 0)
            plsc.store_scatter(cand_v, [idx], v, mask=mk)
            return ptr_vec + plsc.all_reduce_population_count(mk)

        ptr_vec = lax.fori_loop(0, nflag, scan_body,
                                jnp.zeros((L,), jnp.int32))
        c = jnp.max(ptr_vec)
        pad_idx = c + lax.iota(jnp.int32, L)
        plsc.store_scatter(cand_v, [pad_idx],
                           jnp.full((L,), -jnp.inf, jnp.float32))
        nv = (c + (L - 1)) // L

        _to_keys(cand_v, 0, nv)
        stage_v[...] = _bsearch_kth(cand_v, 0, nv, K)
        pltpu.sync_copy(stage_v, thr_hbm.at[r])


_sc_select = functools.partial(
    pl.kernel,
    out_type=jax.ShapeDtypeStruct((B, L), jnp.float32),
    mesh=plsc.VectorSubcoreMesh(core_axis_name="c", subcore_axis_name="s"),
    compiler_params=pltpu.CompilerParams(needs_layout_passes=False),
    scratch_types=[
        pltpu.VMEM((W,), jnp.float32),
        pltpu.VMEM((W,), jnp.float32),
        pltpu.VMEM((W + L,), jnp.float32),
        pltpu.VMEM((NV + L,), jnp.int32),
        pltpu.VMEM((GRID * _CPR + L,), jnp.int32),
        pltpu.VMEM((ROWS_PER * GRID * _CPR,), jnp.float32),
        pltpu.VMEM((L,), jnp.float32),
        pltpu.SemaphoreType.DMA,
        pltpu.SemaphoreType.DMA,
        pltpu.SemaphoreType.DMA,
    ],
)(_sc_body)


def _mask_body(h2_ref, w3t_ref, b3_ref, thr_ref, y_ref):
    j = pl.program_id(0)
    nj = pl.num_programs(0)
    z = _z_tile(h2_ref, w3t_ref, b3_ref, j, nj)
    th = thr_ref[:, 0:1]
    y_ref[...] = jnp.where(z >= th, 1.0, 0.0)


def _apply_mask(h2, w3t, b3r, thr):
    return pl.pallas_call(
        _mask_body,
        grid=(GRID,),
        in_specs=[
            pl.BlockSpec((B, H), lambda j: (0, 0)),
            pl.BlockSpec((H, TILE), lambda j: (0, j)),
            pl.BlockSpec((1, TILE), lambda j: (0, j)),
            pl.BlockSpec((B, L), lambda j: (0, 0)),
        ],
        out_specs=pl.BlockSpec((B, TILE), lambda j: (0, j)),
        out_shape=jax.ShapeDtypeStruct((B, W), jnp.float32),
    )(h2, w3t, b3r, thr)


@jax.jit
def kernel(x, W1, b1, W2, b2, W3, b3):
    b1r = b1.reshape(1, H)
    b2r = b2.reshape(1, H)
    b3r = b3.reshape(1, W)
    w3t = W3.T

    h2 = _compute_h2(x, W1, b1r, W2, b2r)
    z, m128 = _compute_logits(h2, w3t, b3r)
    thr = _sc_select(z, m128.reshape(GRID, B * _CPR))
    return _apply_mask(h2, w3t, b3r, thr)
